# baseline (device time: 9927 ns/iter reference)
import jax
import jax.numpy as jnp
from jax import lax
from jax.experimental import pallas as pl
from jax.experimental.pallas import tpu as pltpu

N_DEV = 32
N_CHUNKS = 8


def kernel(x, w_mat):
    m_per, k = x.shape
    n = w_mat.shape[1]
    n_per = n // N_DEV
    c_w = n // N_CHUNKS
    blk_per_chunk = N_DEV // N_CHUNKS

    def body(x_ref, w_ref, out_ref, y_blocks):
        my = lax.axis_index("i")
        x16 = x_ref[:, :].astype(jnp.bfloat16)
        c1 = 0.7978845608028654
        for c in range(N_CHUNKS):
            w16 = w_ref[:, c * c_w:(c + 1) * c_w].astype(jnp.bfloat16)
            yc = jnp.dot(x16, w16, preferred_element_type=jnp.float32)
            yc = 0.5 * yc * (1.0 + jnp.tanh(c1 * (yc + 0.044715 * yc * yc * yc)))
            y16 = yc.astype(jnp.bfloat16)
            for b in range(blk_per_chunk):
                j = c * blk_per_chunk + b
                y_blocks[j, :, :] = y16[:, b * n_per:(b + 1) * n_per]
        out_ref[pl.ds(my * m_per, m_per), :] = y_blocks[my, :, :].astype(jnp.float32)

    return pl.pallas_call(
        body,
        out_shape=jax.ShapeDtypeStruct((N_DEV * m_per, n_per), jnp.float32),
        in_specs=[
            pl.BlockSpec(memory_space=pltpu.VMEM),
            pl.BlockSpec(memory_space=pltpu.VMEM),
        ],
        out_specs=pl.BlockSpec(memory_space=pltpu.VMEM),
        scratch_shapes=[
            pltpu.VMEM((N_DEV, m_per, n_per), jnp.bfloat16),
        ],
    )(x, w_mat)
